# baseline (speedup 1.0000x reference)
import jax
import jax.numpy as jnp
from jax.experimental import pallas as pl

NZ = 50000
NX = 50000
NC = 50000
DH = 32


def _norm(deg):
    return jnp.where(deg > 0, 1.0 / jnp.sqrt(jnp.maximum(deg, 1.0)), 0.0)


def kernel(z_feats, x_feats, c_feats, edge_index_z2c, edge_index_x2c, eweights_z2c, eweights_x2c, Wz, bz, Wx, bx, Wc, bc, W1, b1, W2, b2, Wo1, bo1, Wo2, bo2, Wo3, bo3):
    src_z, dst_zc = edge_index_z2c[0], edge_index_z2c[1]
    src_x, dst_xc = edge_index_x2c[0], edge_index_x2c[1]

    deg_z = jnp.zeros((NZ,), jnp.float32).at[src_z].add(1.0)
    deg_cz = jnp.zeros((NC,), jnp.float32).at[dst_zc].add(1.0)
    deg_x = jnp.zeros((NX,), jnp.float32).at[src_x].add(1.0)
    deg_cx = jnp.zeros((NC,), jnp.float32).at[dst_xc].add(1.0)
    nz, ncz, nx, ncx = _norm(deg_z), _norm(deg_cz), _norm(deg_x), _norm(deg_cx)

    h_z = jax.nn.relu(z_feats @ Wz + bz)
    h_x = jax.nn.relu(x_feats @ Wx + bx)

    az = (h_z * nz[:, None]) @ W2
    ax = (h_x * nx[:, None]) @ W2
    agg_cz = jnp.zeros((NC, DH), jnp.float32).at[dst_zc].add(az[src_z])
    agg_cx = jnp.zeros((NC, DH), jnp.float32).at[dst_xc].add(ax[src_x])
    h_c = jax.nn.relu(agg_cz * ncz[:, None] + agg_cx * ncx[:, None] + 2.0 * b2)

    cm = (h_c * ncz[:, None]) @ W2
    agg_z = jnp.zeros((NZ, DH), jnp.float32).at[src_z].add(cm[dst_zc])
    h_zo = jax.nn.relu(agg_z * nz[:, None] + b2)

    h = jax.nn.relu(h_zo @ Wo1 + bo1)
    h = jax.nn.relu(h @ Wo2 + bo2)
    logit = h @ Wo3 + bo3
    return jnp.mean(logit, axis=0, keepdims=True)


# speedup vs baseline: 6.0247x; 6.0247x over previous
import jax
import jax.numpy as jnp
from jax import lax
from jax.experimental import pallas as pl
from jax.experimental.pallas import tpu as pltpu
from jax.experimental.pallas import tpu_sc as plsc

NN = 50000
NE = 800000
DH = 32
CHUNK = 128
NE_PAD = 819200
NROWS = NE_PAD // CHUNK
NCORES = 2
NSUB = 16
ROWS_PER_TILE = NROWS // (NCORES * NSUB)
ROWS_PER_TILE_DEG = NROWS // NSUB
NACC = 51200
SLICE = NACC // NSUB
DCHUNK = 160
NHOPS = SLICE // DCHUNK
GG = 25
NG_GS = ROWS_PER_TILE // GG
NG_DEG = ROWS_PER_TILE_DEG // GG

_mesh = plsc.VectorSubcoreMesh(core_axis_name="c", subcore_axis_name="s")
_f32 = jnp.float32
_sc_params = pltpu.CompilerParams(use_tc_tiling_on_sc=False)


def _deg_body(sz, dz, sx, dx, ones_hbm, zeros_hbm, degz, degcz, degx, degcx,
              acc_a, acc_b, idx_a, idx_b, ones_v, stage, sem):
    c = lax.axis_index("c")
    s = lax.axis_index("s")

    pltpu.sync_copy(zeros_hbm, stage)
    pltpu.sync_copy(stage, acc_a.at[pl.ds(s * SLICE, SLICE)])
    pltpu.sync_copy(stage, acc_b.at[pl.ds(s * SLICE, SLICE)])
    pltpu.sync_copy(ones_hbm, ones_v)
    plsc.subcore_barrier()

    base = s * ROWS_PER_TILE_DEG

    def group(g, carry):
        @pl.when(c == 0)
        def _():
            pltpu.sync_copy(sz.at[pl.ds(base + g * GG, GG)], idx_a)
            pltpu.sync_copy(dz.at[pl.ds(base + g * GG, GG)], idx_b)

        @pl.when(c == 1)
        def _():
            pltpu.sync_copy(sx.at[pl.ds(base + g * GG, GG)], idx_a)
            pltpu.sync_copy(dx.at[pl.ds(base + g * GG, GG)], idx_b)

        def body(j, carry2):
            pltpu.sync_copy(ones_v, acc_a.at[idx_a.at[j]], add=True)
            pltpu.sync_copy(ones_v, acc_b.at[idx_b.at[j]], add=True)
            return carry2

        lax.fori_loop(0, GG, body, 0)
        return carry

    lax.fori_loop(0, NG_DEG, group, 0)
    plsc.subcore_barrier()

    def dump(acc, out):
        pltpu.sync_copy(acc.at[pl.ds(s * SLICE, SLICE)], stage)
        pltpu.sync_copy(stage, out.at[pl.ds(s * SLICE, SLICE)])

    @pl.when(c == 0)
    def _():
        dump(acc_a, degz)
        dump(acc_b, degcz)

    @pl.when(c == 1)
    def _():
        dump(acc_a, degx)
        dump(acc_b, degcx)


_deg_kernel = pl.kernel(
    _deg_body,
    out_type=tuple(jax.ShapeDtypeStruct((NACC,), _f32) for _ in range(4)),
    mesh=_mesh,
    scratch_types=[
        pltpu.VMEM_SHARED((NACC,), _f32),
        pltpu.VMEM_SHARED((NACC,), _f32),
        pltpu.VMEM((GG, CHUNK), jnp.int32),
        pltpu.VMEM((GG, CHUNK), jnp.int32),
        pltpu.VMEM((CHUNK,), _f32),
        pltpu.VMEM((SLICE,), _f32),
        pltpu.SemaphoreType.DMA,
    ],
    compiler_params=_sc_params,
)


def _gs_body(table, gidx, sidx, zeros_hbm, out, acc, gv, sv, rows, stage, sem):
    c = lax.axis_index("c")
    s = lax.axis_index("s")

    pltpu.sync_copy(zeros_hbm, stage)

    def init(h, carry):
        pltpu.sync_copy(stage, acc.at[pl.ds(s * SLICE + h * DCHUNK, DCHUNK)])
        return carry

    lax.fori_loop(0, NHOPS, init, 0)
    plsc.subcore_barrier()

    base = (c * NSUB + s) * ROWS_PER_TILE

    def group(g, carry):
        pltpu.sync_copy(gidx.at[pl.ds(base + g * GG, GG)], gv)
        pltpu.sync_copy(sidx.at[pl.ds(base + g * GG, GG)], sv)

        def body(j, carry2):
            pltpu.async_copy(table.at[gv.at[j]], rows, sem).wait()
            pltpu.sync_copy(rows, acc.at[sv.at[j]], add=True)
            return carry2

        lax.fori_loop(0, GG, body, 0)
        return carry

    lax.fori_loop(0, NG_GS, group, 0)
    plsc.subcore_barrier()

    def dump(h, carry):
        pltpu.sync_copy(acc.at[pl.ds(s * SLICE + h * DCHUNK, DCHUNK)], stage)
        pltpu.sync_copy(stage, out.at[c].at[pl.ds(s * SLICE + h * DCHUNK, DCHUNK)])
        return carry

    lax.fori_loop(0, NHOPS, dump, 0)


_gs_kernel = pl.kernel(
    _gs_body,
    out_type=jax.ShapeDtypeStruct((NCORES, NACC, DH), _f32),
    mesh=_mesh,
    scratch_types=[
        pltpu.VMEM_SHARED((NACC, DH), _f32),
        pltpu.VMEM((GG, CHUNK), jnp.int32),
        pltpu.VMEM((GG, CHUNK), jnp.int32),
        pltpu.VMEM((CHUNK, DH), _f32),
        pltpu.VMEM((DCHUNK, DH), _f32),
        pltpu.SemaphoreType.DMA,
    ],
    compiler_params=_sc_params,
)


BLK = 1000
GRID = NN // BLK


def _norm(d):
    return jnp.where(d > 0.0, lax.rsqrt(jnp.maximum(d, 1.0)), 0.0)


def _enc_body(z, x, dgz, dgx, Wz, bz, wx, bx, W2, az, ax):
    nz = _norm(dgz[...])
    nx = _norm(dgx[...])
    hz = jnp.maximum(jnp.dot(z[...], Wz[...], preferred_element_type=_f32) + bz[...], 0.0)
    az[...] = jnp.dot(hz * nz, W2[...], preferred_element_type=_f32)
    hx = jnp.maximum(x[...] * wx[...] + bx[...], 0.0)
    ax[...] = jnp.dot(hx * nx, W2[...], preferred_element_type=_f32)


def _hc_body(a0, a1, x0, x1, dcz, dcx, b2, W2, cm):
    ncz = _norm(dcz[...])
    ncx = _norm(dcx[...])
    hc = jnp.maximum((a0[...] + a1[...]) * ncz + (x0[...] + x1[...]) * ncx
                     + 2.0 * b2[...], 0.0)
    cm[...] = jnp.dot(hc * ncz, W2[...], preferred_element_type=_f32)


def _out_body(p0, p1, dgz, b2, Wo1, bo1, Wo2, bo2, acc):
    nz = _norm(dgz[...])
    hzo = jnp.maximum((p0[...] + p1[...]) * nz + b2[...], 0.0)
    h1 = jnp.maximum(jnp.dot(hzo, Wo1[...], preferred_element_type=_f32) + bo1[...], 0.0)
    h2 = jnp.maximum(jnp.dot(h1, Wo2[...], preferred_element_type=_f32) + bo2[...], 0.0)
    bs = jnp.sum(h2, axis=0, keepdims=True)

    @pl.when(pl.program_id(0) == 0)
    def _():
        acc[...] = jnp.zeros_like(acc)

    acc[...] += bs


def _row_spec(width):
    return pl.BlockSpec((BLK, width), lambda i: (i, 0))


def _full_spec(shape):
    ndim = len(shape)
    return pl.BlockSpec(shape, lambda i: (0,) * ndim)


_enc_call = pl.pallas_call(
    _enc_body,
    grid=(GRID,),
    in_specs=[
        _row_spec(128), _row_spec(1), _row_spec(1), _row_spec(1),
        _full_spec((128, DH)), _full_spec((1, DH)), _full_spec((1, DH)),
        _full_spec((1, DH)), _full_spec((DH, DH)),
    ],
    out_specs=[_row_spec(DH), _row_spec(DH)],
    out_shape=[jax.ShapeDtypeStruct((NN, DH), _f32)] * 2,
)

_hc_call = pl.pallas_call(
    _hc_body,
    grid=(GRID,),
    in_specs=[
        _row_spec(DH), _row_spec(DH), _row_spec(DH), _row_spec(DH),
        _row_spec(1), _row_spec(1),
        _full_spec((1, DH)), _full_spec((DH, DH)),
    ],
    out_specs=_row_spec(DH),
    out_shape=jax.ShapeDtypeStruct((NN, DH), _f32),
)

_out_call = pl.pallas_call(
    _out_body,
    grid=(GRID,),
    in_specs=[
        _row_spec(DH), _row_spec(DH), _row_spec(1),
        _full_spec((1, DH)), _full_spec((DH, DH)), _full_spec((1, DH)),
        _full_spec((DH, DH)), _full_spec((1, DH)),
    ],
    out_specs=pl.BlockSpec((1, DH), lambda i: (0, 0)),
    out_shape=jax.ShapeDtypeStruct((1, DH), _f32),
)


def _pad_idx(idx, fill):
    idx = idx.astype(jnp.int32)
    pad = jnp.full((NE_PAD - NE,), fill, jnp.int32)
    return jnp.concatenate([idx, pad]).reshape(NROWS, CHUNK)


def kernel(z_feats, x_feats, c_feats, edge_index_z2c, edge_index_x2c,
           eweights_z2c, eweights_x2c, Wz, bz, Wx, bx, Wc, bc,
           W1, b1, W2, b2, Wo1, bo1, Wo2, bo2, Wo3, bo3):
    sz = _pad_idx(edge_index_z2c[0], 0)
    dz = _pad_idx(edge_index_z2c[1], NN)
    sx = _pad_idx(edge_index_x2c[0], 0)
    dx = _pad_idx(edge_index_x2c[1], NN)

    ones_hbm = jnp.ones((CHUNK,), _f32)
    zeros1 = jnp.zeros((SLICE,), _f32)
    zeros2 = jnp.zeros((DCHUNK, DH), _f32)

    degz, degcz, degx, degcx = (v[:NN] for v in
                                _deg_kernel(sz, dz, sx, dx, ones_hbm, zeros1))

    col = lambda v: v.reshape(NN, 1)
    row = lambda v: v.reshape(1, DH)

    az, ax = _enc_call(z_feats, x_feats, col(degz), col(degx),
                       Wz, row(bz), Wx.reshape(1, DH), row(bx), W2)

    parts_cz = _gs_kernel(az, sz, dz, zeros2)
    parts_cx = _gs_kernel(ax, sx, dx, zeros2)

    cm = _hc_call(parts_cz[0, :NN], parts_cz[1, :NN],
                  parts_cx[0, :NN], parts_cx[1, :NN],
                  col(degcz), col(degcx), row(b2), W2)

    parts_z = _gs_kernel(cm, dz, sz, zeros2)

    acc = _out_call(parts_z[0, :NN], parts_z[1, :NN], col(degz),
                    row(b2), Wo1, row(bo1), Wo2, row(bo2))

    return acc @ Wo3 * (1.0 / NN) + bo3[None, :]


# speedup vs baseline: 10.8337x; 1.7982x over previous
import jax
import jax.numpy as jnp
from jax import lax
from jax.experimental import pallas as pl
from jax.experimental.pallas import tpu as pltpu
from jax.experimental.pallas import tpu_sc as plsc

NN = 50000
NE = 800000
DH = 32
CHUNK = 128
NE_PAD = 819200
NROWS = NE_PAD // CHUNK
NCORES = 2
NSUB = 16
RPT_FULL = NROWS // NSUB
RPT_HALF = NROWS // (NCORES * NSUB)
NACC = 51200
SLICE = NACC // NSUB
DCHUNK = 160
NHOPS = SLICE // DCHUNK
GG = 20

_mesh = plsc.VectorSubcoreMesh(core_axis_name="c", subcore_axis_name="s")
_f32 = jnp.float32
_sc_params = pltpu.CompilerParams(use_tc_tiling_on_sc=False)


def _init_acc(acc, zeros_hbm, stage, s):
    pltpu.sync_copy(zeros_hbm, stage)

    def init(h, carry):
        pltpu.sync_copy(stage, acc.at[pl.ds(s * SLICE + h * DCHUNK, DCHUNK)])
        return carry

    lax.fori_loop(0, NHOPS, init, 0)


def _dump_acc(acc, out, stage, s):
    def dump(h, carry):
        pltpu.sync_copy(acc.at[pl.ds(s * SLICE + h * DCHUNK, DCHUNK)], stage)
        pltpu.sync_copy(stage, out.at[pl.ds(s * SLICE + h * DCHUNK, DCHUNK)])
        return carry

    lax.fori_loop(0, NHOPS, dump, 0)


def _gs_loop(table, gidx, sidx, acc, gv, sv, r0, r1, sem0, sem1, base, nrows):
    def group(g, carry):
        pltpu.sync_copy(gidx.at[pl.ds(base + g * GG, GG)], gv)
        pltpu.sync_copy(sidx.at[pl.ds(base + g * GG, GG)], sv)
        pltpu.async_copy(table.at[gv.at[0]], r0, sem0)

        def body(j, carry2):
            @pl.when(j % 2 == 0)
            def _():
                @pl.when(j + 1 < GG)
                def _():
                    pltpu.async_copy(table.at[gv.at[j + 1]], r1, sem1)
                pltpu.make_async_copy(table.at[gv.at[j]], r0, sem0).wait()
                pltpu.sync_copy(r0, acc.at[sv.at[j]], add=True)

            @pl.when(j % 2 == 1)
            def _():
                @pl.when(j + 1 < GG)
                def _():
                    pltpu.async_copy(table.at[gv.at[j + 1]], r0, sem0)
                pltpu.make_async_copy(table.at[gv.at[j]], r1, sem1).wait()
                pltpu.sync_copy(r1, acc.at[sv.at[j]], add=True)

            return carry2

        lax.fori_loop(0, GG, body, 0)
        return carry

    lax.fori_loop(0, nrows // GG, group, 0)


def _fwd_body(az, ax, sz, dz, sx, dx, zeros_hbm, out,
              acc, gv, sv, r0, r1, stage, sem0, sem1):
    c = lax.axis_index("c")
    s = lax.axis_index("s")
    _init_acc(acc, zeros_hbm, stage, s)
    plsc.subcore_barrier()

    @pl.when(c == 0)
    def _():
        _gs_loop(az, sz, dz, acc, gv, sv, r0, r1, sem0, sem1,
                 s * RPT_FULL, RPT_FULL)

    @pl.when(c == 1)
    def _():
        _gs_loop(ax, sx, dx, acc, gv, sv, r0, r1, sem0, sem1,
                 s * RPT_FULL, RPT_FULL)

    plsc.subcore_barrier()
    _dump_acc(acc, out.at[c], stage, s)


def _bwd_body(cm, dz, sz, zeros_hbm, out,
              acc, gv, sv, r0, r1, stage, sem0, sem1):
    c = lax.axis_index("c")
    s = lax.axis_index("s")
    _init_acc(acc, zeros_hbm, stage, s)
    plsc.subcore_barrier()
    _gs_loop(cm, dz, sz, acc, gv, sv, r0, r1, sem0, sem1,
             (c * NSUB + s) * RPT_HALF, RPT_HALF)
    plsc.subcore_barrier()
    _dump_acc(acc, out.at[c], stage, s)


_gs_scratch = [
    pltpu.VMEM_SHARED((NACC, DH), _f32),
    pltpu.VMEM((GG, CHUNK), jnp.int32),
    pltpu.VMEM((GG, CHUNK), jnp.int32),
    pltpu.VMEM((CHUNK, DH), _f32),
    pltpu.VMEM((CHUNK, DH), _f32),
    pltpu.VMEM((DCHUNK, DH), _f32),
    pltpu.SemaphoreType.DMA,
    pltpu.SemaphoreType.DMA,
]

_fwd_kernel = pl.kernel(
    _fwd_body,
    out_type=jax.ShapeDtypeStruct((NCORES, NACC, DH), _f32),
    mesh=_mesh,
    scratch_types=_gs_scratch,
    compiler_params=_sc_params,
)

_bwd_kernel = pl.kernel(
    _bwd_body,
    out_type=jax.ShapeDtypeStruct((NCORES, NACC, DH), _f32),
    mesh=_mesh,
    scratch_types=_gs_scratch,
    compiler_params=_sc_params,
)


def _deg_body(sz, dz, sx, dx, ones_hbm, zeros_hbm, degz, degcz, degx, degcx,
              acc_a, acc_b, idx_a, idx_b, ones_v, stage, sem):
    c = lax.axis_index("c")
    s = lax.axis_index("s")

    pltpu.sync_copy(zeros_hbm, stage)
    pltpu.sync_copy(stage, acc_a.at[pl.ds(s * SLICE, SLICE)])
    pltpu.sync_copy(stage, acc_b.at[pl.ds(s * SLICE, SLICE)])
    pltpu.sync_copy(ones_hbm, ones_v)
    plsc.subcore_barrier()

    base = s * RPT_FULL

    def run(gi, si):
        def group(g, carry):
            pltpu.sync_copy(gi.at[pl.ds(base + g * GG, GG)], idx_a)
            pltpu.sync_copy(si.at[pl.ds(base + g * GG, GG)], idx_b)

            def body(j, carry2):
                pltpu.sync_copy(ones_v, acc_a.at[idx_a.at[j]], add=True)
                pltpu.sync_copy(ones_v, acc_b.at[idx_b.at[j]], add=True)
                return carry2

            lax.fori_loop(0, GG, body, 0)
            return carry

        lax.fori_loop(0, RPT_FULL // GG, group, 0)

    @pl.when(c == 0)
    def _():
        run(sz, dz)

    @pl.when(c == 1)
    def _():
        run(sx, dx)

    plsc.subcore_barrier()

    def dump(acc, out):
        pltpu.sync_copy(acc.at[pl.ds(s * SLICE, SLICE)], stage)
        pltpu.sync_copy(stage, out.at[pl.ds(s * SLICE, SLICE)])

    @pl.when(c == 0)
    def _():
        dump(acc_a, degz)
        dump(acc_b, degcz)

    @pl.when(c == 1)
    def _():
        dump(acc_a, degx)
        dump(acc_b, degcx)


_deg_kernel = pl.kernel(
    _deg_body,
    out_type=tuple(jax.ShapeDtypeStruct((NACC,), _f32) for _ in range(4)),
    mesh=_mesh,
    scratch_types=[
        pltpu.VMEM_SHARED((NACC,), _f32),
        pltpu.VMEM_SHARED((NACC,), _f32),
        pltpu.VMEM((GG, CHUNK), jnp.int32),
        pltpu.VMEM((GG, CHUNK), jnp.int32),
        pltpu.VMEM((CHUNK,), _f32),
        pltpu.VMEM((SLICE,), _f32),
        pltpu.SemaphoreType.DMA,
    ],
    compiler_params=_sc_params,
)


BLK = 1000
GRID = NN // BLK


def _norm(d):
    return jnp.where(d > 0.0, lax.rsqrt(jnp.maximum(d, 1.0)), 0.0)


def _enc_body(z, x, dgz, dgx, Wz, bz, wx, bx, W2, az, ax):
    nz = _norm(dgz[...])
    nx = _norm(dgx[...])
    hz = jnp.maximum(jnp.dot(z[...], Wz[...], preferred_element_type=_f32) + bz[...], 0.0)
    az[...] = jnp.dot(hz * nz, W2[...], preferred_element_type=_f32)
    hx = jnp.maximum(x[...] * wx[...] + bx[...], 0.0)
    ax[...] = jnp.dot(hx * nx, W2[...], preferred_element_type=_f32)


def _hc_body(a0, a1, dcz, dcx, b2, W2, cm):
    ncz = _norm(dcz[...])
    ncx = _norm(dcx[...])
    hc = jnp.maximum(a0[...] * ncz + a1[...] * ncx + 2.0 * b2[...], 0.0)
    cm[...] = jnp.dot(hc * ncz, W2[...], preferred_element_type=_f32)


def _out_body(p0, p1, dgz, b2, Wo1, bo1, Wo2, bo2, acc):
    nz = _norm(dgz[...])
    hzo = jnp.maximum((p0[...] + p1[...]) * nz + b2[...], 0.0)
    h1 = jnp.maximum(jnp.dot(hzo, Wo1[...], preferred_element_type=_f32) + bo1[...], 0.0)
    h2 = jnp.maximum(jnp.dot(h1, Wo2[...], preferred_element_type=_f32) + bo2[...], 0.0)
    bs = jnp.sum(h2, axis=0, keepdims=True)

    @pl.when(pl.program_id(0) == 0)
    def _():
        acc[...] = jnp.zeros_like(acc)

    acc[...] += bs


def _row_spec(width):
    return pl.BlockSpec((BLK, width), lambda i: (i, 0))


def _full_spec(shape):
    ndim = len(shape)
    return pl.BlockSpec(shape, lambda i: (0,) * ndim)


_enc_call = pl.pallas_call(
    _enc_body,
    grid=(GRID,),
    in_specs=[
        _row_spec(128), _row_spec(1), _row_spec(1), _row_spec(1),
        _full_spec((128, DH)), _full_spec((1, DH)), _full_spec((1, DH)),
        _full_spec((1, DH)), _full_spec((DH, DH)),
    ],
    out_specs=[_row_spec(DH), _row_spec(DH)],
    out_shape=[jax.ShapeDtypeStruct((NN, DH), _f32)] * 2,
)

_hc_call = pl.pallas_call(
    _hc_body,
    grid=(GRID,),
    in_specs=[
        _row_spec(DH), _row_spec(DH),
        _row_spec(1), _row_spec(1),
        _full_spec((1, DH)), _full_spec((DH, DH)),
    ],
    out_specs=_row_spec(DH),
    out_shape=jax.ShapeDtypeStruct((NN, DH), _f32),
)

_out_call = pl.pallas_call(
    _out_body,
    grid=(GRID,),
    in_specs=[
        _row_spec(DH), _row_spec(DH), _row_spec(1),
        _full_spec((1, DH)), _full_spec((DH, DH)), _full_spec((1, DH)),
        _full_spec((DH, DH)), _full_spec((1, DH)),
    ],
    out_specs=pl.BlockSpec((1, DH), lambda i: (0, 0)),
    out_shape=jax.ShapeDtypeStruct((1, DH), _f32),
)


def _pad_idx(idx):
    idx = idx.astype(jnp.int32)
    pad = NN + (jnp.arange(NE_PAD - NE, dtype=jnp.int32) % (NACC - NN))
    return jnp.concatenate([idx, pad]).reshape(NROWS, CHUNK)


def _pad_rows(t):
    return jnp.pad(t, ((0, NACC - NN), (0, 0)))


def kernel(z_feats, x_feats, c_feats, edge_index_z2c, edge_index_x2c,
           eweights_z2c, eweights_x2c, Wz, bz, Wx, bx, Wc, bc,
           W1, b1, W2, b2, Wo1, bo1, Wo2, bo2, Wo3, bo3):
    sz = _pad_idx(edge_index_z2c[0])
    dz = _pad_idx(edge_index_z2c[1])
    sx = _pad_idx(edge_index_x2c[0])
    dx = _pad_idx(edge_index_x2c[1])

    ones_hbm = jnp.ones((CHUNK,), _f32)
    zeros1 = jnp.zeros((SLICE,), _f32)
    zeros2 = jnp.zeros((DCHUNK, DH), _f32)

    degz, degcz, degx, degcx = (v[:NN] for v in
                                _deg_kernel(sz, dz, sx, dx, ones_hbm, zeros1))

    col = lambda v: v.reshape(NN, 1)
    row = lambda v: v.reshape(1, DH)

    az, ax = _enc_call(z_feats, x_feats, col(degz), col(degx),
                       Wz, row(bz), Wx.reshape(1, DH), row(bx), W2)

    agg_c = _fwd_kernel(_pad_rows(az), _pad_rows(ax), sz, dz, sx, dx, zeros2)

    cm = _hc_call(agg_c[0, :NN], agg_c[1, :NN],
                  col(degcz), col(degcx), row(b2), W2)

    parts_z = _bwd_kernel(_pad_rows(cm), dz, sz, zeros2)

    acc = _out_call(parts_z[0, :NN], parts_z[1, :NN], col(degz),
                    row(b2), Wo1, row(bo1), Wo2, row(bo2))

    return acc @ Wo3 * (1.0 / NN) + bo3[None, :]


# speedup vs baseline: 12.9739x; 1.1975x over previous
import jax
import jax.numpy as jnp
from jax import lax
from jax.experimental import pallas as pl
from jax.experimental.pallas import tpu as pltpu
from jax.experimental.pallas import tpu_sc as plsc

NN = 50000
NE = 800000
DH = 32
CHUNK = 128
NE_PAD = 819200
NROWS = NE_PAD // CHUNK
NCORES = 2
NSUB = 16
RPT_FULL = NROWS // NSUB
RPT_HALF = NROWS // (NCORES * NSUB)
NACC = 51200
SLICE = NACC // NSUB
DCHUNK = 160
NHOPS = SLICE // DCHUNK
GG = 20

_mesh = plsc.VectorSubcoreMesh(core_axis_name="c", subcore_axis_name="s")
_f32 = jnp.float32
_sc_params = pltpu.CompilerParams(use_tc_tiling_on_sc=False)


def _init_acc(acc, zeros_hbm, stage, s):
    pltpu.sync_copy(zeros_hbm, stage)

    def init(h, carry):
        pltpu.sync_copy(stage, acc.at[pl.ds(s * SLICE + h * DCHUNK, DCHUNK)])
        return carry

    lax.fori_loop(0, NHOPS, init, 0)


def _dump_acc(acc, out, stage, s):
    def dump(h, carry):
        pltpu.sync_copy(acc.at[pl.ds(s * SLICE + h * DCHUNK, DCHUNK)], stage)
        pltpu.sync_copy(stage, out.at[pl.ds(s * SLICE + h * DCHUNK, DCHUNK)])
        return carry

    lax.fori_loop(0, NHOPS, dump, 0)


def _gs_loop(table, gidx, sidx, acc, gv, sv, r0, r1, sg0, sg1, ss0, ss1,
             base, nrows):
    def group(g, carry):
        pltpu.sync_copy(gidx.at[pl.ds(base + g * GG, GG)], gv)
        pltpu.sync_copy(sidx.at[pl.ds(base + g * GG, GG)], sv)
        pltpu.async_copy(table.at[gv.at[0]], r0, sg0)

        def body(j, carry2):
            @pl.when(j % 2 == 0)
            def _():
                @pl.when(j + 1 < GG)
                def _():
                    @pl.when(j > 0)
                    def _():
                        pltpu.make_async_copy(r1, acc.at[sv.at[j]], ss1).wait()
                    pltpu.async_copy(table.at[gv.at[j + 1]], r1, sg1)
                pltpu.make_async_copy(table.at[gv.at[j]], r0, sg0).wait()
                pltpu.async_copy(r0, acc.at[sv.at[j]], ss0, add=True)

            @pl.when(j % 2 == 1)
            def _():
                @pl.when(j + 1 < GG)
                def _():
                    pltpu.make_async_copy(r0, acc.at[sv.at[j]], ss0).wait()
                    pltpu.async_copy(table.at[gv.at[j + 1]], r0, sg0)
                pltpu.make_async_copy(table.at[gv.at[j]], r1, sg1).wait()
                pltpu.async_copy(r1, acc.at[sv.at[j]], ss1, add=True)

            return carry2

        lax.fori_loop(0, GG, body, 0)
        pltpu.make_async_copy(r0, acc.at[sv.at[0]], ss0).wait()
        pltpu.make_async_copy(r1, acc.at[sv.at[0]], ss1).wait()
        return carry

    lax.fori_loop(0, nrows // GG, group, 0)


def _fwd_body(az, ax, sz, dz, sx, dx, zeros_hbm, out_z, out_x,
              acc, gv, sv, r0, r1, stage, sg0, sg1, ss0, ss1):
    c = lax.axis_index("c")
    s = lax.axis_index("s")
    _init_acc(acc, zeros_hbm, stage, s)
    plsc.subcore_barrier()

    @pl.when(c == 0)
    def _():
        _gs_loop(az, sz, dz, acc, gv, sv, r0, r1, sg0, sg1, ss0, ss1,
                 s * RPT_FULL, RPT_FULL)

    @pl.when(c == 1)
    def _():
        _gs_loop(ax, sx, dx, acc, gv, sv, r0, r1, sg0, sg1, ss0, ss1,
                 s * RPT_FULL, RPT_FULL)

    plsc.subcore_barrier()

    @pl.when(c == 0)
    def _():
        _dump_acc(acc, out_z, stage, s)

    @pl.when(c == 1)
    def _():
        _dump_acc(acc, out_x, stage, s)


def _bwd_body(cm, dz, sz, zeros_hbm, out0, out1,
              acc, gv, sv, r0, r1, stage, sg0, sg1, ss0, ss1):
    c = lax.axis_index("c")
    s = lax.axis_index("s")
    _init_acc(acc, zeros_hbm, stage, s)
    plsc.subcore_barrier()
    _gs_loop(cm, dz, sz, acc, gv, sv, r0, r1, sg0, sg1, ss0, ss1,
             (c * NSUB + s) * RPT_HALF, RPT_HALF)
    plsc.subcore_barrier()

    @pl.when(c == 0)
    def _():
        _dump_acc(acc, out0, stage, s)

    @pl.when(c == 1)
    def _():
        _dump_acc(acc, out1, stage, s)


_gs_scratch = [
    pltpu.VMEM_SHARED((NACC, DH), _f32),
    pltpu.VMEM((GG, CHUNK), jnp.int32),
    pltpu.VMEM((GG, CHUNK), jnp.int32),
    pltpu.VMEM((CHUNK, DH), _f32),
    pltpu.VMEM((CHUNK, DH), _f32),
    pltpu.VMEM((DCHUNK, DH), _f32),
    pltpu.SemaphoreType.DMA,
    pltpu.SemaphoreType.DMA,
    pltpu.SemaphoreType.DMA,
    pltpu.SemaphoreType.DMA,
]

_acc_out = jax.ShapeDtypeStruct((NACC, DH), _f32)

_fwd_kernel = pl.kernel(
    _fwd_body,
    out_type=(_acc_out, _acc_out),
    mesh=_mesh,
    scratch_types=_gs_scratch,
    compiler_params=_sc_params,
)

_bwd_kernel = pl.kernel(
    _bwd_body,
    out_type=(_acc_out, _acc_out),
    mesh=_mesh,
    scratch_types=_gs_scratch,
    compiler_params=_sc_params,
)


def _deg_body(sz, dz, sx, dx, ones_hbm, zeros_hbm, degz, degcz, degx, degcx,
              acc_a, acc_b, idx_a, idx_b, ones_v, stage, sem):
    c = lax.axis_index("c")
    s = lax.axis_index("s")

    pltpu.sync_copy(zeros_hbm, stage)
    pltpu.sync_copy(stage, acc_a.at[pl.ds(s * SLICE, SLICE)])
    pltpu.sync_copy(stage, acc_b.at[pl.ds(s * SLICE, SLICE)])
    pltpu.sync_copy(ones_hbm, ones_v)
    plsc.subcore_barrier()

    base = s * RPT_FULL

    def run(gi, si):
        def group(g, carry):
            pltpu.sync_copy(gi.at[pl.ds(base + g * GG, GG)], idx_a)
            pltpu.sync_copy(si.at[pl.ds(base + g * GG, GG)], idx_b)

            def fire(j, carry2):
                pltpu.async_copy(ones_v, acc_a.at[idx_a.at[j]], sem, add=True)
                pltpu.async_copy(ones_v, acc_b.at[idx_b.at[j]], sem, add=True)
                return carry2

            lax.fori_loop(0, GG, fire, 0)

            def drain(j, carry2):
                pltpu.make_async_copy(ones_v, acc_a.at[idx_a.at[0]], sem).wait()
                pltpu.make_async_copy(ones_v, acc_b.at[idx_b.at[0]], sem).wait()
                return carry2

            lax.fori_loop(0, GG, drain, 0)
            return carry

        lax.fori_loop(0, RPT_FULL // GG, group, 0)

    @pl.when(c == 0)
    def _():
        run(sz, dz)

    @pl.when(c == 1)
    def _():
        run(sx, dx)

    plsc.subcore_barrier()

    def dump(acc, out):
        pltpu.sync_copy(acc.at[pl.ds(s * SLICE, SLICE)], stage)
        pltpu.sync_copy(stage, out.at[pl.ds(s * SLICE, SLICE)])

    @pl.when(c == 0)
    def _():
        dump(acc_a, degz)
        dump(acc_b, degcz)

    @pl.when(c == 1)
    def _():
        dump(acc_a, degx)
        dump(acc_b, degcx)


_deg_kernel = pl.kernel(
    _deg_body,
    out_type=tuple(jax.ShapeDtypeStruct((NACC,), _f32) for _ in range(4)),
    mesh=_mesh,
    scratch_types=[
        pltpu.VMEM_SHARED((NACC,), _f32),
        pltpu.VMEM_SHARED((NACC,), _f32),
        pltpu.VMEM((GG, CHUNK), jnp.int32),
        pltpu.VMEM((GG, CHUNK), jnp.int32),
        pltpu.VMEM((CHUNK,), _f32),
        pltpu.VMEM((SLICE,), _f32),
        pltpu.SemaphoreType.DMA,
    ],
    compiler_params=_sc_params,
)


BLK = 1000
GRID = NN // BLK


def _norm(d):
    return jnp.where(d > 0.0, lax.rsqrt(jnp.maximum(d, 1.0)), 0.0)


def _enc_body(z, x, dgz, dgx, Wz, bz, wx, bx, W2, az, ax):
    nz = _norm(dgz[...])
    nx = _norm(dgx[...])
    hz = jnp.maximum(jnp.dot(z[...], Wz[...], preferred_element_type=_f32) + bz[...], 0.0)
    az[...] = jnp.dot(hz * nz, W2[...], preferred_element_type=_f32)
    hx = jnp.maximum(x[...] * wx[...] + bx[...], 0.0)
    ax[...] = jnp.dot(hx * nx, W2[...], preferred_element_type=_f32)


def _hc_body(a0, a1, dcz, dcx, b2, W2, cm):
    ncz = _norm(dcz[...])
    ncx = _norm(dcx[...])
    hc = jnp.maximum(a0[...] * ncz + a1[...] * ncx + 2.0 * b2[...], 0.0)
    cm[...] = jnp.dot(hc * ncz, W2[...], preferred_element_type=_f32)


def _out_body(p0, p1, dgz, b2, Wo1, bo1, Wo2, bo2, acc):
    nz = _norm(dgz[...])
    hzo = jnp.maximum((p0[...] + p1[...]) * nz + b2[...], 0.0)
    h1 = jnp.maximum(jnp.dot(hzo, Wo1[...], preferred_element_type=_f32) + bo1[...], 0.0)
    h2 = jnp.maximum(jnp.dot(h1, Wo2[...], preferred_element_type=_f32) + bo2[...], 0.0)
    bs = jnp.sum(h2, axis=0, keepdims=True)

    @pl.when(pl.program_id(0) == 0)
    def _():
        acc[...] = jnp.zeros_like(acc)

    acc[...] += bs


def _row_spec(width):
    return pl.BlockSpec((BLK, width), lambda i: (i, 0))


def _full_spec(shape):
    ndim = len(shape)
    return pl.BlockSpec(shape, lambda i: (0,) * ndim)


_enc_call = pl.pallas_call(
    _enc_body,
    grid=(GRID,),
    in_specs=[
        _row_spec(128), _row_spec(1), _row_spec(1), _row_spec(1),
        _full_spec((128, DH)), _full_spec((1, DH)), _full_spec((1, DH)),
        _full_spec((1, DH)), _full_spec((DH, DH)),
    ],
    out_specs=[_row_spec(DH), _row_spec(DH)],
    out_shape=[jax.ShapeDtypeStruct((NACC, DH), _f32)] * 2,
)

_hc_call = pl.pallas_call(
    _hc_body,
    grid=(GRID,),
    in_specs=[
        _row_spec(DH), _row_spec(DH),
        _row_spec(1), _row_spec(1),
        _full_spec((1, DH)), _full_spec((DH, DH)),
    ],
    out_specs=_row_spec(DH),
    out_shape=jax.ShapeDtypeStruct((NACC, DH), _f32),
)

_out_call = pl.pallas_call(
    _out_body,
    grid=(GRID,),
    in_specs=[
        _row_spec(DH), _row_spec(DH), _row_spec(1),
        _full_spec((1, DH)), _full_spec((DH, DH)), _full_spec((1, DH)),
        _full_spec((DH, DH)), _full_spec((1, DH)),
    ],
    out_specs=pl.BlockSpec((1, DH), lambda i: (0, 0)),
    out_shape=jax.ShapeDtypeStruct((1, DH), _f32),
)


def _pad_idx(idx):
    idx = idx.astype(jnp.int32)
    pad = NN + (jnp.arange(NE_PAD - NE, dtype=jnp.int32) % (NACC - NN))
    return jnp.concatenate([idx, pad]).reshape(NROWS, CHUNK)


def kernel(z_feats, x_feats, c_feats, edge_index_z2c, edge_index_x2c,
           eweights_z2c, eweights_x2c, Wz, bz, Wx, bx, Wc, bc,
           W1, b1, W2, b2, Wo1, bo1, Wo2, bo2, Wo3, bo3):
    sz = _pad_idx(edge_index_z2c[0])
    dz = _pad_idx(edge_index_z2c[1])
    sx = _pad_idx(edge_index_x2c[0])
    dx = _pad_idx(edge_index_x2c[1])

    ones_hbm = jnp.ones((CHUNK,), _f32)
    zeros1 = jnp.zeros((SLICE,), _f32)
    zeros2 = jnp.zeros((DCHUNK, DH), _f32)

    degz, degcz, degx, degcx = _deg_kernel(sz, dz, sx, dx, ones_hbm, zeros1)

    col = lambda v: v.reshape(NACC, 1)
    row = lambda v: v.reshape(1, DH)

    az, ax = _enc_call(z_feats, x_feats, col(degz), col(degx),
                       Wz, row(bz), Wx.reshape(1, DH), row(bx), W2)

    agg_cz, agg_cx = _fwd_kernel(az, ax, sz, dz, sx, dx, zeros2)

    cm = _hc_call(agg_cz, agg_cx, col(degcz), col(degcx), row(b2), W2)

    p0, p1 = _bwd_kernel(cm, dz, sz, zeros2)

    acc = _out_call(p0, p1, col(degz), row(b2), Wo1, row(bo1), Wo2, row(bo2))

    return acc @ Wo3 * (1.0 / NN) + bo3[None, :]


# speedup vs baseline: 14.8144x; 1.1419x over previous
import jax
import jax.numpy as jnp
from jax import lax
from jax.experimental import pallas as pl
from jax.experimental.pallas import tpu as pltpu
from jax.experimental.pallas import tpu_sc as plsc

NN = 50000
NE = 800000
DH = 32
CHUNK = 128
NROWS = NE // CHUNK
NCORES = 2
NSUB = 16
GG = 25
NGRP = NROWS // GG
NACC = 51200
SLICE = NACC // NSUB
DCHUNK = 160
NHOPS = SLICE // DCHUNK

_mesh = plsc.VectorSubcoreMesh(core_axis_name="c", subcore_axis_name="s")
_f32 = jnp.float32
_sc_params = pltpu.CompilerParams(use_tc_tiling_on_sc=False)


def _init_acc(acc, zeros_hbm, stage, s):
    pltpu.sync_copy(zeros_hbm, stage)

    def init(h, carry):
        pltpu.sync_copy(stage, acc.at[pl.ds(s * SLICE + h * DCHUNK, DCHUNK)])
        return carry

    lax.fori_loop(0, NHOPS, init, 0)


def _dump_acc(acc, out, stage, s):
    def dump(h, carry):
        pltpu.sync_copy(acc.at[pl.ds(s * SLICE + h * DCHUNK, DCHUNK)], stage)
        pltpu.sync_copy(stage, out.at[pl.ds(s * SLICE + h * DCHUNK, DCHUNK)])
        return carry

    lax.fori_loop(0, NHOPS, dump, 0)


def _gs_loop(table, gidx, sidx, acc, gv, sv, r0, r1, sg0, sg1, ss0, ss1,
             w, stride, kmax):
    def group(k, carry):
        g = w + stride * k

        @pl.when(g < NGRP)
        def _():
            pltpu.sync_copy(gidx.at[pl.ds(g * GG, GG)], gv)
            pltpu.sync_copy(sidx.at[pl.ds(g * GG, GG)], sv)
            pltpu.async_copy(table.at[gv.at[0]], r0, sg0)

            def body(j, carry2):
                @pl.when(j % 2 == 0)
                def _():
                    @pl.when(j + 1 < GG)
                    def _():
                        @pl.when(j > 0)
                        def _():
                            pltpu.make_async_copy(r1, acc.at[sv.at[j]], ss1).wait()
                        pltpu.async_copy(table.at[gv.at[j + 1]], r1, sg1)
                    pltpu.make_async_copy(table.at[gv.at[j]], r0, sg0).wait()
                    pltpu.async_copy(r0, acc.at[sv.at[j]], ss0, add=True)

                @pl.when(j % 2 == 1)
                def _():
                    @pl.when(j + 1 < GG)
                    def _():
                        pltpu.make_async_copy(r0, acc.at[sv.at[j]], ss0).wait()
                        pltpu.async_copy(table.at[gv.at[j + 1]], r0, sg0)
                    pltpu.make_async_copy(table.at[gv.at[j]], r1, sg1).wait()
                    pltpu.async_copy(r1, acc.at[sv.at[j]], ss1, add=True)

                return carry2

            lax.fori_loop(0, GG, body, 0)
            pltpu.make_async_copy(r0, acc.at[sv.at[0]], ss0).wait()
            pltpu.make_async_copy(r1, acc.at[sv.at[0]], ss1).wait()

        return carry

    lax.fori_loop(0, kmax, group, 0)


def _fwd_body(az, ax, sz, dz, sx, dx, zeros_hbm, out_z, out_x,
              acc, gv, sv, r0, r1, stage, sg0, sg1, ss0, ss1):
    c = lax.axis_index("c")
    s = lax.axis_index("s")
    _init_acc(acc, zeros_hbm, stage, s)
    plsc.subcore_barrier()

    @pl.when(c == 0)
    def _():
        _gs_loop(az, sz, dz, acc, gv, sv, r0, r1, sg0, sg1, ss0, ss1,
                 s, NSUB, NGRP // NSUB + 1)

    @pl.when(c == 1)
    def _():
        _gs_loop(ax, sx, dx, acc, gv, sv, r0, r1, sg0, sg1, ss0, ss1,
                 s, NSUB, NGRP // NSUB + 1)

    plsc.subcore_barrier()

    @pl.when(c == 0)
    def _():
        _dump_acc(acc, out_z, stage, s)

    @pl.when(c == 1)
    def _():
        _dump_acc(acc, out_x, stage, s)


def _bwd_body(cm, dz, sz, zeros_hbm, out0, out1,
              acc, gv, sv, r0, r1, stage, sg0, sg1, ss0, ss1):
    c = lax.axis_index("c")
    s = lax.axis_index("s")
    _init_acc(acc, zeros_hbm, stage, s)
    plsc.subcore_barrier()
    _gs_loop(cm, dz, sz, acc, gv, sv, r0, r1, sg0, sg1, ss0, ss1,
             c * NSUB + s, NCORES * NSUB, NGRP // (NCORES * NSUB) + 1)
    plsc.subcore_barrier()

    @pl.when(c == 0)
    def _():
        _dump_acc(acc, out0, stage, s)

    @pl.when(c == 1)
    def _():
        _dump_acc(acc, out1, stage, s)


_gs_scratch = [
    pltpu.VMEM_SHARED((NACC, DH), _f32),
    pltpu.VMEM((GG, CHUNK), jnp.int32),
    pltpu.VMEM((GG, CHUNK), jnp.int32),
    pltpu.VMEM((CHUNK, DH), _f32),
    pltpu.VMEM((CHUNK, DH), _f32),
    pltpu.VMEM((DCHUNK, DH), _f32),
    pltpu.SemaphoreType.DMA,
    pltpu.SemaphoreType.DMA,
    pltpu.SemaphoreType.DMA,
    pltpu.SemaphoreType.DMA,
]

_acc_out = jax.ShapeDtypeStruct((NACC, DH), _f32)

_fwd_kernel = pl.kernel(
    _fwd_body,
    out_type=(_acc_out, _acc_out),
    mesh=_mesh,
    scratch_types=_gs_scratch,
    compiler_params=_sc_params,
)

_bwd_kernel = pl.kernel(
    _bwd_body,
    out_type=(_acc_out, _acc_out),
    mesh=_mesh,
    scratch_types=_gs_scratch,
    compiler_params=_sc_params,
)


def _deg_body(sz, dz, sx, dx, ones_hbm, zeros_hbm, degz, degcz, degx, degcx,
              acc_a, acc_b, idx_a, idx_b, ones_v, stage, sem):
    c = lax.axis_index("c")
    s = lax.axis_index("s")

    pltpu.sync_copy(zeros_hbm, stage)
    pltpu.sync_copy(stage, acc_a.at[pl.ds(s * SLICE, SLICE)])
    pltpu.sync_copy(stage, acc_b.at[pl.ds(s * SLICE, SLICE)])
    pltpu.sync_copy(ones_hbm, ones_v)
    plsc.subcore_barrier()

    def run(gi, si):
        def group(k, carry):
            g = s + NSUB * k

            @pl.when(g < NGRP)
            def _():
                pltpu.sync_copy(gi.at[pl.ds(g * GG, GG)], idx_a)
                pltpu.sync_copy(si.at[pl.ds(g * GG, GG)], idx_b)

                def fire(j, carry2):
                    pltpu.async_copy(ones_v, acc_a.at[idx_a.at[j]], sem, add=True)
                    pltpu.async_copy(ones_v, acc_b.at[idx_b.at[j]], sem, add=True)
                    return carry2

                lax.fori_loop(0, GG, fire, 0)

                def drain(j, carry2):
                    pltpu.make_async_copy(ones_v, acc_a.at[idx_a.at[0]], sem).wait()
                    pltpu.make_async_copy(ones_v, acc_b.at[idx_b.at[0]], sem).wait()
                    return carry2

                lax.fori_loop(0, GG, drain, 0)

            return carry

        lax.fori_loop(0, NGRP // NSUB + 1, group, 0)

    @pl.when(c == 0)
    def _():
        run(sz, dz)

    @pl.when(c == 1)
    def _():
        run(sx, dx)

    plsc.subcore_barrier()

    def dump(acc, out):
        pltpu.sync_copy(acc.at[pl.ds(s * SLICE, SLICE)], stage)
        pltpu.sync_copy(stage, out.at[pl.ds(s * SLICE, SLICE)])

    @pl.when(c == 0)
    def _():
        dump(acc_a, degz)
        dump(acc_b, degcz)

    @pl.when(c == 1)
    def _():
        dump(acc_a, degx)
        dump(acc_b, degcx)


_deg_kernel = pl.kernel(
    _deg_body,
    out_type=tuple(jax.ShapeDtypeStruct((NACC,), _f32) for _ in range(4)),
    mesh=_mesh,
    scratch_types=[
        pltpu.VMEM_SHARED((NACC,), _f32),
        pltpu.VMEM_SHARED((NACC,), _f32),
        pltpu.VMEM((GG, CHUNK), jnp.int32),
        pltpu.VMEM((GG, CHUNK), jnp.int32),
        pltpu.VMEM((CHUNK,), _f32),
        pltpu.VMEM((SLICE,), _f32),
        pltpu.SemaphoreType.DMA,
    ],
    compiler_params=_sc_params,
)


EBLK = 1000
EGRID = NN // EBLK
BLK = 2048
GRID = NACC // BLK


def _norm(d):
    return jnp.where(d > 0.0, lax.rsqrt(jnp.maximum(d, 1.0)), 0.0)


def _enc0_body(z, Wz, bz, W2, az0):
    hz = jnp.maximum(jnp.dot(z[...], Wz[...], preferred_element_type=_f32) + bz[...], 0.0)
    az0[...] = jnp.dot(hz, W2[...], preferred_element_type=_f32)


def _scale_body(az0, x, dgz, dgx, wx, bx, W2, az, ax):
    nz = _norm(dgz[...]).reshape(BLK, 1)
    nx = _norm(dgx[...]).reshape(BLK, 1)
    az[...] = az0[...] * nz
    hx = jnp.maximum(x[...].reshape(BLK, 1) * wx[...] + bx[...], 0.0)
    ax[...] = jnp.dot(hx * nx, W2[...], preferred_element_type=_f32)


def _hc_body(a0, a1, dcz, dcx, b2, W2, cm):
    ncz = _norm(dcz[...]).reshape(BLK, 1)
    ncx = _norm(dcx[...]).reshape(BLK, 1)
    hc = jnp.maximum(a0[...] * ncz + a1[...] * ncx + 2.0 * b2[...], 0.0)
    cm[...] = jnp.dot(hc * ncz, W2[...], preferred_element_type=_f32)


def _out_body(p0, p1, dgz, b2, Wo1, bo1, Wo2, bo2, acc):
    nz = _norm(dgz[...]).reshape(BLK, 1)
    hzo = jnp.maximum((p0[...] + p1[...]) * nz + b2[...], 0.0)
    h1 = jnp.maximum(jnp.dot(hzo, Wo1[...], preferred_element_type=_f32) + bo1[...], 0.0)
    h2 = jnp.maximum(jnp.dot(h1, Wo2[...], preferred_element_type=_f32) + bo2[...], 0.0)
    rid = pl.program_id(0) * BLK + lax.broadcasted_iota(jnp.int32, (BLK, 1), 0)
    h2 = jnp.where(rid < NN, h2, 0.0)
    bs = jnp.sum(h2, axis=0, keepdims=True)

    @pl.when(pl.program_id(0) == 0)
    def _():
        acc[...] = jnp.zeros_like(acc)

    acc[...] += bs


def _t_spec():
    return pl.BlockSpec((BLK, DH), lambda i: (i, 0))


def _v_spec():
    return pl.BlockSpec((BLK,), lambda i: (i,))


def _full_spec(shape):
    ndim = len(shape)
    return pl.BlockSpec(shape, lambda i: (0,) * ndim)


_table = jax.ShapeDtypeStruct((NACC, DH), _f32)

_enc0_call = pl.pallas_call(
    _enc0_body,
    grid=(EGRID,),
    in_specs=[
        pl.BlockSpec((EBLK, 128), lambda i: (i, 0)),
        _full_spec((128, DH)), _full_spec((1, DH)), _full_spec((DH, DH)),
    ],
    out_specs=pl.BlockSpec((EBLK, DH), lambda i: (i, 0)),
    out_shape=_table,
)

_scale_call = pl.pallas_call(
    _scale_body,
    grid=(GRID,),
    in_specs=[
        _t_spec(), _v_spec(), _v_spec(), _v_spec(),
        _full_spec((1, DH)), _full_spec((1, DH)), _full_spec((DH, DH)),
    ],
    out_specs=[_t_spec(), _t_spec()],
    out_shape=[_table] * 2,
)

_hc_call = pl.pallas_call(
    _hc_body,
    grid=(GRID,),
    in_specs=[
        _t_spec(), _t_spec(), _v_spec(), _v_spec(),
        _full_spec((1, DH)), _full_spec((DH, DH)),
    ],
    out_specs=_t_spec(),
    out_shape=_table,
)

_out_call = pl.pallas_call(
    _out_body,
    grid=(GRID,),
    in_specs=[
        _t_spec(), _t_spec(), _v_spec(),
        _full_spec((1, DH)), _full_spec((DH, DH)), _full_spec((1, DH)),
        _full_spec((DH, DH)), _full_spec((1, DH)),
    ],
    out_specs=pl.BlockSpec((1, DH), lambda i: (0, 0)),
    out_shape=jax.ShapeDtypeStruct((1, DH), _f32),
)


def _rows(idx):
    return idx.astype(jnp.int32).reshape(NROWS, CHUNK)


def kernel(z_feats, x_feats, c_feats, edge_index_z2c, edge_index_x2c,
           eweights_z2c, eweights_x2c, Wz, bz, Wx, bx, Wc, bc,
           W1, b1, W2, b2, Wo1, bo1, Wo2, bo2, Wo3, bo3):
    sz = _rows(edge_index_z2c[0])
    dz = _rows(edge_index_z2c[1])
    sx = _rows(edge_index_x2c[0])
    dx = _rows(edge_index_x2c[1])

    ones_hbm = jnp.ones((CHUNK,), _f32)
    zeros1 = jnp.zeros((SLICE,), _f32)
    zeros2 = jnp.zeros((DCHUNK, DH), _f32)
    x_pad = jnp.pad(x_feats.reshape(NN), (0, NACC - NN))

    degz, degcz, degx, degcx = _deg_kernel(sz, dz, sx, dx, ones_hbm, zeros1)

    row = lambda v: v.reshape(1, DH)

    az0 = _enc0_call(z_feats, Wz, row(bz), W2)
    az, ax = _scale_call(az0, x_pad, degz, degx,
                         Wx.reshape(1, DH), row(bx), W2)

    agg_cz, agg_cx = _fwd_kernel(az, ax, sz, dz, sx, dx, zeros2)

    cm = _hc_call(agg_cz, agg_cx, degcz, degcx, row(b2), W2)

    p0, p1 = _bwd_kernel(cm, dz, sz, zeros2)

    acc = _out_call(p0, p1, degz, row(b2), Wo1, row(bo1), Wo2, row(bo2))

    return acc @ Wo3 * (1.0 / NN) + bo3[None, :]


# speedup vs baseline: 15.5869x; 1.0521x over previous
import jax
import jax.numpy as jnp
from jax import lax
from jax.experimental import pallas as pl
from jax.experimental.pallas import tpu as pltpu
from jax.experimental.pallas import tpu_sc as plsc

NN = 50000
NE = 800000
DH = 32
CHUNK = 128
NROWS = NE // CHUNK
NCORES = 2
NSUB = 16
GG = 25
NGRP = NROWS // GG
NACC = 51200
SLICE = NACC // NSUB
DCHUNK = 160
NHOPS = SLICE // DCHUNK

_mesh = plsc.VectorSubcoreMesh(core_axis_name="c", subcore_axis_name="s")
_f32 = jnp.float32
_bf16 = jnp.bfloat16
_sc_params = pltpu.CompilerParams(use_tc_tiling_on_sc=False)


def _init_acc(acc, zeros_hbm, stage, s):
    pltpu.sync_copy(zeros_hbm, stage)

    def init(h, carry):
        pltpu.sync_copy(stage, acc.at[pl.ds(s * SLICE + h * DCHUNK, DCHUNK)])
        return carry

    lax.fori_loop(0, NHOPS, init, 0)


def _dump_acc(acc, out, stage, s):
    def dump(h, carry):
        pltpu.sync_copy(acc.at[pl.ds(s * SLICE + h * DCHUNK, DCHUNK)], stage)
        pltpu.sync_copy(stage, out.at[pl.ds(s * SLICE + h * DCHUNK, DCHUNK)])
        return carry

    lax.fori_loop(0, NHOPS, dump, 0)


def _gs_loop(table, gidx, sidx, acc, gv, sv, r0, r1, sg0, sg1, ss0, ss1,
             w, stride, kmax):
    def group(k, carry):
        g = w + stride * k

        @pl.when(g < NGRP)
        def _():
            pltpu.sync_copy(gidx.at[pl.ds(g * GG, GG)], gv)
            pltpu.sync_copy(sidx.at[pl.ds(g * GG, GG)], sv)
            pltpu.async_copy(table.at[gv.at[0]], r0, sg0)

            def body(j, carry2):
                @pl.when(j % 2 == 0)
                def _():
                    @pl.when(j + 1 < GG)
                    def _():
                        @pl.when(j > 0)
                        def _():
                            pltpu.make_async_copy(r1, acc.at[sv.at[j]], ss1).wait()
                        pltpu.async_copy(table.at[gv.at[j + 1]], r1, sg1)
                    pltpu.make_async_copy(table.at[gv.at[j]], r0, sg0).wait()
                    pltpu.async_copy(r0, acc.at[sv.at[j]], ss0, add=True)

                @pl.when(j % 2 == 1)
                def _():
                    @pl.when(j + 1 < GG)
                    def _():
                        pltpu.make_async_copy(r0, acc.at[sv.at[j]], ss0).wait()
                        pltpu.async_copy(table.at[gv.at[j + 1]], r0, sg0)
                    pltpu.make_async_copy(table.at[gv.at[j]], r1, sg1).wait()
                    pltpu.async_copy(r1, acc.at[sv.at[j]], ss1, add=True)

                return carry2

            lax.fori_loop(0, GG, body, 0)
            pltpu.make_async_copy(r0, acc.at[sv.at[0]], ss0).wait()
            pltpu.make_async_copy(r1, acc.at[sv.at[0]], ss1).wait()

        return carry

    lax.fori_loop(0, kmax, group, 0)


def _fwd_body(az, ax, sz, dz, sx, dx, zeros_hbm, out_z, out_x,
              acc, gv, sv, r0, r1, stage, sg0, sg1, ss0, ss1):
    c = lax.axis_index("c")
    s = lax.axis_index("s")
    _init_acc(acc, zeros_hbm, stage, s)
    plsc.subcore_barrier()

    @pl.when(c == 0)
    def _():
        _gs_loop(az, sz, dz, acc, gv, sv, r0, r1, sg0, sg1, ss0, ss1,
                 s, NSUB, NGRP // NSUB + 1)

    @pl.when(c == 1)
    def _():
        _gs_loop(ax, sx, dx, acc, gv, sv, r0, r1, sg0, sg1, ss0, ss1,
                 s, NSUB, NGRP // NSUB + 1)

    plsc.subcore_barrier()

    @pl.when(c == 0)
    def _():
        _dump_acc(acc, out_z, stage, s)

    @pl.when(c == 1)
    def _():
        _dump_acc(acc, out_x, stage, s)


def _bwd_body(cm, dz, sz, zeros_hbm, out0, out1,
              acc, gv, sv, r0, r1, stage, sg0, sg1, ss0, ss1):
    c = lax.axis_index("c")
    s = lax.axis_index("s")
    _init_acc(acc, zeros_hbm, stage, s)
    plsc.subcore_barrier()
    _gs_loop(cm, dz, sz, acc, gv, sv, r0, r1, sg0, sg1, ss0, ss1,
             c * NSUB + s, NCORES * NSUB, NGRP // (NCORES * NSUB) + 1)
    plsc.subcore_barrier()

    @pl.when(c == 0)
    def _():
        _dump_acc(acc, out0, stage, s)

    @pl.when(c == 1)
    def _():
        _dump_acc(acc, out1, stage, s)


_gs_scratch = [
    pltpu.VMEM_SHARED((NACC, DH), _bf16),
    pltpu.VMEM((GG, CHUNK), jnp.int32),
    pltpu.VMEM((GG, CHUNK), jnp.int32),
    pltpu.VMEM((CHUNK, DH), _bf16),
    pltpu.VMEM((CHUNK, DH), _bf16),
    pltpu.VMEM((DCHUNK, DH), _bf16),
    pltpu.SemaphoreType.DMA,
    pltpu.SemaphoreType.DMA,
    pltpu.SemaphoreType.DMA,
    pltpu.SemaphoreType.DMA,
]

_acc_out = jax.ShapeDtypeStruct((NACC, DH), _bf16)

_fwd_kernel = pl.kernel(
    _fwd_body,
    out_type=(_acc_out, _acc_out),
    mesh=_mesh,
    scratch_types=_gs_scratch,
    compiler_params=_sc_params,
)

_bwd_kernel = pl.kernel(
    _bwd_body,
    out_type=(_acc_out, _acc_out),
    mesh=_mesh,
    scratch_types=_gs_scratch,
    compiler_params=_sc_params,
)


def _deg_body(sz, dz, sx, dx, ones_hbm, zeros_hbm, degz, degcz, degx, degcx,
              acc_a, acc_b, idx_a, idx_b, ones_v, stage, sem):
    c = lax.axis_index("c")
    s = lax.axis_index("s")

    pltpu.sync_copy(zeros_hbm, stage)
    pltpu.sync_copy(stage, acc_a.at[pl.ds(s * SLICE, SLICE)])
    pltpu.sync_copy(stage, acc_b.at[pl.ds(s * SLICE, SLICE)])
    pltpu.sync_copy(ones_hbm, ones_v)
    plsc.subcore_barrier()

    def run(gi, si):
        def group(k, carry):
            g = s + NSUB * k

            @pl.when(g < NGRP)
            def _():
                pltpu.sync_copy(gi.at[pl.ds(g * GG, GG)], idx_a)
                pltpu.sync_copy(si.at[pl.ds(g * GG, GG)], idx_b)

                def fire(j, carry2):
                    pltpu.async_copy(ones_v, acc_a.at[idx_a.at[j]], sem, add=True)
                    pltpu.async_copy(ones_v, acc_b.at[idx_b.at[j]], sem, add=True)
                    return carry2

                lax.fori_loop(0, GG, fire, 0)

                def drain(j, carry2):
                    pltpu.make_async_copy(ones_v, acc_a.at[idx_a.at[0]], sem).wait()
                    pltpu.make_async_copy(ones_v, acc_b.at[idx_b.at[0]], sem).wait()
                    return carry2

                lax.fori_loop(0, GG, drain, 0)

            return carry

        lax.fori_loop(0, NGRP // NSUB + 1, group, 0)

    @pl.when(c == 0)
    def _():
        run(sz, dz)

    @pl.when(c == 1)
    def _():
        run(sx, dx)

    plsc.subcore_barrier()

    def dump(acc, out):
        pltpu.sync_copy(acc.at[pl.ds(s * SLICE, SLICE)], stage)
        pltpu.sync_copy(stage, out.at[pl.ds(s * SLICE, SLICE)])

    @pl.when(c == 0)
    def _():
        dump(acc_a, degz)
        dump(acc_b, degcz)

    @pl.when(c == 1)
    def _():
        dump(acc_a, degx)
        dump(acc_b, degcx)


_deg_kernel = pl.kernel(
    _deg_body,
    out_type=tuple(jax.ShapeDtypeStruct((NACC,), _f32) for _ in range(4)),
    mesh=_mesh,
    scratch_types=[
        pltpu.VMEM_SHARED((NACC,), _f32),
        pltpu.VMEM_SHARED((NACC,), _f32),
        pltpu.VMEM((GG, CHUNK), jnp.int32),
        pltpu.VMEM((GG, CHUNK), jnp.int32),
        pltpu.VMEM((CHUNK,), _f32),
        pltpu.VMEM((SLICE,), _f32),
        pltpu.SemaphoreType.DMA,
    ],
    compiler_params=_sc_params,
)


EBLK = 1000
EGRID = NN // EBLK
BLK = 2048
GRID = NACC // BLK


def _norm(d):
    return jnp.where(d > 0.0, lax.rsqrt(jnp.maximum(d, 1.0)), 0.0)


def _enc0_body(z, Wz, bz, W2, az0):
    hz = jnp.maximum(jnp.dot(z[...], Wz[...], preferred_element_type=_f32) + bz[...], 0.0)
    az0[...] = jnp.dot(hz, W2[...], preferred_element_type=_f32).astype(_bf16)


def _scale_body(az0, x, dgz, dgx, wx, bx, W2, az, ax):
    nz = _norm(dgz[...]).reshape(BLK, 1)
    nx = _norm(dgx[...]).reshape(BLK, 1)
    az[...] = (az0[...].astype(_f32) * nz).astype(_bf16)
    hx = jnp.maximum(x[...].reshape(BLK, 1) * wx[...] + bx[...], 0.0)
    ax[...] = jnp.dot(hx * nx, W2[...], preferred_element_type=_f32).astype(_bf16)


def _hc_body(a0, a1, dcz, dcx, b2, W2, cm):
    ncz = _norm(dcz[...]).reshape(BLK, 1)
    ncx = _norm(dcx[...]).reshape(BLK, 1)
    hc = jnp.maximum(a0[...].astype(_f32) * ncz + a1[...].astype(_f32) * ncx
                     + 2.0 * b2[...], 0.0)
    cm[...] = jnp.dot(hc * ncz, W2[...], preferred_element_type=_f32).astype(_bf16)


def _out_body(p0, p1, dgz, b2, Wo1, bo1, Wo2, bo2, acc):
    nz = _norm(dgz[...]).reshape(BLK, 1)
    hzo = jnp.maximum((p0[...].astype(_f32) + p1[...].astype(_f32)) * nz + b2[...], 0.0)
    h1 = jnp.maximum(jnp.dot(hzo, Wo1[...], preferred_element_type=_f32) + bo1[...], 0.0)
    h2 = jnp.maximum(jnp.dot(h1, Wo2[...], preferred_element_type=_f32) + bo2[...], 0.0)
    rid = pl.program_id(0) * BLK + lax.broadcasted_iota(jnp.int32, (BLK, 1), 0)
    h2 = jnp.where(rid < NN, h2, 0.0)
    bs = jnp.sum(h2, axis=0, keepdims=True)

    @pl.when(pl.program_id(0) == 0)
    def _():
        acc[...] = jnp.zeros_like(acc)

    acc[...] += bs


def _t_spec():
    return pl.BlockSpec((BLK, DH), lambda i: (i, 0))


def _v_spec():
    return pl.BlockSpec((BLK,), lambda i: (i,))


def _full_spec(shape):
    ndim = len(shape)
    return pl.BlockSpec(shape, lambda i: (0,) * ndim)


_table = jax.ShapeDtypeStruct((NACC, DH), _bf16)

_enc0_call = pl.pallas_call(
    _enc0_body,
    grid=(EGRID,),
    in_specs=[
        pl.BlockSpec((EBLK, 128), lambda i: (i, 0)),
        _full_spec((128, DH)), _full_spec((1, DH)), _full_spec((DH, DH)),
    ],
    out_specs=pl.BlockSpec((EBLK, DH), lambda i: (i, 0)),
    out_shape=_table,
)

_scale_call = pl.pallas_call(
    _scale_body,
    grid=(GRID,),
    in_specs=[
        _t_spec(), _v_spec(), _v_spec(), _v_spec(),
        _full_spec((1, DH)), _full_spec((1, DH)), _full_spec((DH, DH)),
    ],
    out_specs=[_t_spec(), _t_spec()],
    out_shape=[_table] * 2,
)

_hc_call = pl.pallas_call(
    _hc_body,
    grid=(GRID,),
    in_specs=[
        _t_spec(), _t_spec(), _v_spec(), _v_spec(),
        _full_spec((1, DH)), _full_spec((DH, DH)),
    ],
    out_specs=_t_spec(),
    out_shape=_table,
)

_out_call = pl.pallas_call(
    _out_body,
    grid=(GRID,),
    in_specs=[
        _t_spec(), _t_spec(), _v_spec(),
        _full_spec((1, DH)), _full_spec((DH, DH)), _full_spec((1, DH)),
        _full_spec((DH, DH)), _full_spec((1, DH)),
    ],
    out_specs=pl.BlockSpec((1, DH), lambda i: (0, 0)),
    out_shape=jax.ShapeDtypeStruct((1, DH), _f32),
)


def _rows(idx):
    return idx.astype(jnp.int32).reshape(NROWS, CHUNK)


def kernel(z_feats, x_feats, c_feats, edge_index_z2c, edge_index_x2c,
           eweights_z2c, eweights_x2c, Wz, bz, Wx, bx, Wc, bc,
           W1, b1, W2, b2, Wo1, bo1, Wo2, bo2, Wo3, bo3):
    sz = _rows(edge_index_z2c[0])
    dz = _rows(edge_index_z2c[1])
    sx = _rows(edge_index_x2c[0])
    dx = _rows(edge_index_x2c[1])

    ones_hbm = jnp.ones((CHUNK,), _f32)
    zeros1 = jnp.zeros((SLICE,), _f32)
    zeros2 = jnp.zeros((DCHUNK, DH), _bf16)
    x_pad = jnp.pad(x_feats.reshape(NN), (0, NACC - NN))

    degz, degcz, degx, degcx = _deg_kernel(sz, dz, sx, dx, ones_hbm, zeros1)

    row = lambda v: v.reshape(1, DH)

    az0 = _enc0_call(z_feats, Wz, row(bz), W2)
    az, ax = _scale_call(az0, x_pad, degz, degx,
                         Wx.reshape(1, DH), row(bx), W2)

    agg_cz, agg_cx = _fwd_kernel(az, ax, sz, dz, sx, dx, zeros2)

    cm = _hc_call(agg_cz, agg_cx, degcz, degcx, row(b2), W2)

    p0, p1 = _bwd_kernel(cm, dz, sz, zeros2)

    acc = _out_call(p0, p1, degz, row(b2), Wo1, row(bo1), Wo2, row(bo2))

    return acc @ Wo3 * (1.0 / NN) + bo3[None, :]
